# baseline (speedup 1.0000x reference)
import jax
import jax.numpy as jnp
from jax import lax
from jax.experimental import pallas as pl
from jax.experimental.pallas import tpu as pltpu
from jax.experimental.pallas import tpu_sc as plsc

_BATCH = 16384
_NUM_DOMAINS = 100
_LANES = 16
_NUM_CORES = 1
_NUM_SUBCORES = 16
_NUM_WORKERS = _NUM_CORES * _NUM_SUBCORES
_CHUNK = _BATCH // _NUM_WORKERS
_WPAD = 112


def _sc_body(idx_hbm, x_hbm, w_hbm, out_hbm, idx_v, x_v, w_v, sem_a):
    base = lax.axis_index("s") * _CHUNK
    cw = pltpu.async_copy(w_hbm, w_v.at[pl.ds(0, _NUM_DOMAINS)], sem_a)
    ci = pltpu.async_copy(idx_hbm.at[pl.ds(base, _CHUNK)], idx_v, sem_a)
    cx = pltpu.async_copy(x_hbm.at[pl.ds(base, _CHUNK)], x_v, sem_a)
    cw.wait()
    ci.wait()
    cx.wait()
    zero = jnp.zeros((_LANES,), jnp.float32)
    for j in range(_WPAD // _LANES):
        sl = pl.ds(j * _LANES, _LANES)
        w_v[sl] = jnp.maximum(w_v[sl], zero)

    @plsc.parallel_loop(0, _CHUNK, _LANES, unroll=2)
    def _gather_mul(i):
        sl = pl.ds(i, _LANES)
        x_v[sl] = x_v[sl] * plsc.load_gather(w_v, [idx_v[sl]])

    pltpu.sync_copy(x_v, out_hbm.at[pl.ds(base, _CHUNK)])


def kernel(idxes, x, raw_weights):
    idxes = idxes.astype(jnp.int32)
    x_flat = x.reshape(-1)
    mesh = plsc.VectorSubcoreMesh(
        core_axis_name="c", subcore_axis_name="s", num_cores=_NUM_CORES
    )
    out = pl.kernel(
        _sc_body,
        out_type=jax.ShapeDtypeStruct((_BATCH,), jnp.float32),
        mesh=mesh,
        compiler_params=pltpu.CompilerParams(needs_layout_passes=False),
        scratch_types=[
            pltpu.VMEM((_CHUNK,), jnp.int32),
            pltpu.VMEM((_CHUNK,), jnp.float32),
            pltpu.VMEM((_WPAD,), jnp.float32),
            pltpu.SemaphoreType.DMA,
        ],
    )(idxes, x_flat, raw_weights)
    return out.reshape(_BATCH, 1)


# speedup vs baseline: 1.0052x; 1.0052x over previous
import jax
import jax.numpy as jnp
from jax import lax
from jax.experimental import pallas as pl
from jax.experimental.pallas import tpu as pltpu
from jax.experimental.pallas import tpu_sc as plsc

_BATCH = 16384
_NUM_DOMAINS = 100
_LANES = 16
_NUM_CORES = 1
_NUM_SUBCORES = 16
_NUM_WORKERS = _NUM_CORES * _NUM_SUBCORES
_CHUNK = _BATCH // _NUM_WORKERS


_HALF = _CHUNK // 2
_WPAD = 112


def _sc_body(idx_hbm, x_hbm, w_hbm, out_hbm, idx_v, x_v, w_v, out_v,
             sem_a, sem_b, sem_o):
    base = lax.axis_index("s") * _CHUNK
    lo, hi = pl.ds(0, _HALF), pl.ds(_HALF, _HALF)
    cw = pltpu.async_copy(w_hbm, w_v.at[pl.ds(0, _NUM_DOMAINS)], sem_a)
    ci0 = pltpu.async_copy(idx_hbm.at[pl.ds(base, _HALF)], idx_v.at[lo], sem_a)
    cx0 = pltpu.async_copy(x_hbm.at[pl.ds(base, _HALF)], x_v.at[lo], sem_a)
    ci1 = pltpu.async_copy(
        idx_hbm.at[pl.ds(base + _HALF, _HALF)], idx_v.at[hi], sem_b)
    cx1 = pltpu.async_copy(
        x_hbm.at[pl.ds(base + _HALF, _HALF)], x_v.at[hi], sem_b)
    cw.wait()
    ci0.wait()
    cx0.wait()
    zero = jnp.zeros((_LANES,), jnp.float32)
    for j in range(_WPAD // _LANES):
        sl = pl.ds(j * _LANES, _LANES)
        w_v[sl] = jnp.maximum(w_v[sl], zero)
    @plsc.parallel_loop(0, _HALF, _LANES, unroll=4)
    def _first_half(i):
        sl = pl.ds(i, _LANES)
        out_v[sl] = x_v[sl] * plsc.load_gather(w_v, [idx_v[sl]])

    co0 = pltpu.async_copy(out_v.at[lo], out_hbm.at[pl.ds(base, _HALF)], sem_o)
    ci1.wait()
    cx1.wait()

    @plsc.parallel_loop(_HALF, _CHUNK, _LANES, unroll=4)
    def _second_half(i):
        sl = pl.ds(i, _LANES)
        out_v[sl] = x_v[sl] * plsc.load_gather(w_v, [idx_v[sl]])
    co1 = pltpu.async_copy(
        out_v.at[hi], out_hbm.at[pl.ds(base + _HALF, _HALF)], sem_o)
    co0.wait()
    co1.wait()


def kernel(idxes, x, raw_weights):
    idxes = idxes.astype(jnp.int32)
    x_flat = x.reshape(-1)
    mesh = plsc.VectorSubcoreMesh(
        core_axis_name="c", subcore_axis_name="s", num_cores=_NUM_CORES
    )
    out = pl.kernel(
        _sc_body,
        out_type=jax.ShapeDtypeStruct((_BATCH,), jnp.float32),
        mesh=mesh,
        compiler_params=pltpu.CompilerParams(needs_layout_passes=False),
        scratch_types=[
            pltpu.VMEM((_CHUNK,), jnp.int32),
            pltpu.VMEM((_CHUNK,), jnp.float32),
            pltpu.VMEM((_WPAD,), jnp.float32),
            pltpu.VMEM((_CHUNK,), jnp.float32),
            pltpu.SemaphoreType.DMA,
            pltpu.SemaphoreType.DMA,
            pltpu.SemaphoreType.DMA,
        ],
    )(idxes, x_flat, raw_weights)
    return out.reshape(_BATCH, 1)
